# Initial kernel scaffold; baseline (speedup 1.0000x reference)
import jax
import jax.numpy as jnp
from jax.experimental import pallas as pl


def kernel(x):
    raise NotImplementedError("write your pallas kernel here")


# speedup vs baseline: 3.7943x; 3.7943x over previous
import functools

import jax
import jax.numpy as jnp
from jax import lax
from jax.experimental import pallas as pl
from jax.experimental.pallas import tpu as pltpu
from jax.experimental.pallas import tpu_sc as plsc

_V = 1000
_F = 10
_ROWS = 1024 * 50

_info = plsc.get_sparse_core_info()
_NC, _NS, _L = _info.num_cores, _info.num_subcores, _info.num_lanes
_NW = _NC * _NS
_RPW = _ROWS // _NW
_C = 80
_CHUNKS = _RPW // _C
_WORDS = _C * _V
_XW = _C * _F

_mesh = plsc.VectorSubcoreMesh(core_axis_name="c", subcore_axis_name="s")


@functools.partial(
    pl.kernel,
    out_type=jax.ShapeDtypeStruct((_ROWS * _V,), jnp.float32),
    mesh=_mesh,
    scratch_types=[
        pltpu.VMEM((_WORDS,), jnp.float32),
        pltpu.VMEM((_XW,), jnp.int32),
    ],
    compiler_params=pltpu.CompilerParams(needs_layout_passes=False),
)
def _ten_hot(x_hbm, out_hbm, buf, xv):
    wid = lax.axis_index("s") * _NC + lax.axis_index("c")
    base_row = wid * _RPW

    zeros = jnp.zeros((_L,), jnp.float32)
    ones = jnp.ones((_L,), jnp.float32)
    lane = lax.iota(jnp.int32, _L)

    def zero_body(j, carry):
        buf[pl.ds(j * _L, _L)] = zeros
        return carry

    lax.fori_loop(0, _WORDS // _L, zero_body, 0)

    def flat_idx(i):
        e = lane + i * _L
        row = lax.shift_right_logical(e * 52429, 19)
        val = xv[pl.ds(i * _L, _L)]
        return row * _V + val

    def chunk_body(g, carry):
        row0 = base_row + g * _C
        pltpu.sync_copy(x_hbm.at[pl.ds(row0 * _F, _XW)], xv)

        def scat(i, c):
            plsc.store_scatter(buf, [flat_idx(i)], ones)
            return c

        lax.fori_loop(0, _XW // _L, scat, 0)
        pltpu.sync_copy(buf, out_hbm.at[pl.ds(row0 * _V, _WORDS)])

        def clean(i, c):
            plsc.store_scatter(buf, [flat_idx(i)], zeros)
            return c

        lax.fori_loop(0, _XW // _L, clean, 0)
        return carry

    lax.fori_loop(0, _CHUNKS, chunk_body, 0)


def kernel(x):
    n, t, f = x.shape
    out = _ten_hot(x.reshape(-1))
    return out.reshape(n, t, _V)


# speedup vs baseline: 3.9199x; 1.0331x over previous
import functools

import jax
import jax.numpy as jnp
from jax import lax
from jax.experimental import pallas as pl
from jax.experimental.pallas import tpu as pltpu
from jax.experimental.pallas import tpu_sc as plsc

_V = 1000
_F = 10
_ROWS = 1024 * 50

_info = plsc.get_sparse_core_info()
_NC, _NS, _L = _info.num_cores, _info.num_subcores, _info.num_lanes
_NW = _NC * _NS
_RPW = _ROWS // _NW
_C = 40
_CHUNKS = _RPW // _C
_PAIRS = _CHUNKS // 2
_WORDS = _C * _V
_XC = _C * _F
_XW = _RPW * _F

_mesh = plsc.VectorSubcoreMesh(core_axis_name="c", subcore_axis_name="s")


@functools.partial(
    pl.kernel,
    out_type=jax.ShapeDtypeStruct((_ROWS * _V,), jnp.float32),
    mesh=_mesh,
    scratch_types=[
        pltpu.VMEM((_WORDS,), jnp.float32),
        pltpu.VMEM((_WORDS,), jnp.float32),
        pltpu.VMEM((_XW,), jnp.int32),
        pltpu.SemaphoreType.DMA,
        pltpu.SemaphoreType.DMA,
    ],
    compiler_params=pltpu.CompilerParams(needs_layout_passes=False),
)
def _ten_hot(x_hbm, out_hbm, buf0, buf1, xv, sem0, sem1):
    wid = lax.axis_index("s") * _NC + lax.axis_index("c")
    base_row = wid * _RPW

    zeros = jnp.zeros((_L,), jnp.float32)
    ones = jnp.ones((_L,), jnp.float32)
    lane = lax.iota(jnp.int32, _L)

    pltpu.sync_copy(x_hbm.at[pl.ds(base_row * _F, _XW)], xv)

    for buf in (buf0, buf1):
        def zero_body(j, carry, buf=buf):
            buf[pl.ds(j * _L, _L)] = zeros
            return carry

        lax.fori_loop(0, _WORDS // _L, zero_body, 0)

    def scatter_chunk(buf, g, val):
        xoff = g * _XC

        def body(i, carry):
            e = lane + i * _L
            row = lax.shift_right_logical(e * 52429, 19)
            v = xv[pl.ds(xoff + i * _L, _L)]
            plsc.store_scatter(buf, [row * _V + v], val)
            return carry

        lax.fori_loop(0, _XC // _L, body, 0)

    def issue_out(buf, sem, g):
        row0 = base_row + g * _C
        pltpu.async_copy(buf, out_hbm.at[pl.ds(row0 * _V, _WORDS)], sem)

    for b, (buf, sem) in enumerate(((buf0, sem0), (buf1, sem1))):
        scatter_chunk(buf, b, ones)
        issue_out(buf, sem, b)

    def pair_body(j, carry):
        for b, (buf, sem) in enumerate(((buf0, sem0), (buf1, sem1))):
            g = 2 * j + b
            pltpu.make_async_copy(
                buf, out_hbm.at[pl.ds(0, _WORDS)], sem
            ).wait()
            scatter_chunk(buf, g - 2, zeros)
            scatter_chunk(buf, g, ones)
            issue_out(buf, sem, g)
        return carry

    lax.fori_loop(1, _PAIRS, pair_body, 0)

    for buf, sem in ((buf0, sem0), (buf1, sem1)):
        pltpu.make_async_copy(buf, out_hbm.at[pl.ds(0, _WORDS)], sem).wait()


def kernel(x):
    n, t, f = x.shape
    out = _ten_hot(x.reshape(-1))
    return out.reshape(n, t, _V)


# speedup vs baseline: 6.7322x; 1.7174x over previous
import functools

import jax
import jax.numpy as jnp
from jax import lax
from jax.experimental import pallas as pl
from jax.experimental.pallas import tpu as pltpu
from jax.experimental.pallas import tpu_sc as plsc

_N = 1024
_T = 50
_V = 1000
_F = 10

_info = plsc.get_sparse_core_info()
_NC, _NS, _L = _info.num_cores, _info.num_subcores, _info.num_lanes
_NW = _NC * _NS
_NPW = _N // _NW
_XSLAB = _T * _F
_XW = _NPW * _XSLAB
_SVEC = (_XSLAB + _L - 1) // _L
_ZVEC = (_V + _L - 1) // _L

_mesh = plsc.VectorSubcoreMesh(core_axis_name="c", subcore_axis_name="s")


@functools.partial(
    pl.kernel,
    out_type=jax.ShapeDtypeStruct((_N, _T, _V), jnp.float32),
    mesh=_mesh,
    scratch_types=[
        pltpu.VMEM((_T, _V), jnp.float32),
        pltpu.VMEM((_T, _V), jnp.float32),
        pltpu.VMEM((_XW,), jnp.int32),
        pltpu.SemaphoreType.DMA,
        pltpu.SemaphoreType.DMA,
    ],
    compiler_params=pltpu.CompilerParams(needs_layout_passes=False),
)
def _ten_hot(x_hbm, out_hbm, buf0, buf1, xv, sem0, sem1):
    wid = lax.axis_index("s") * _NC + lax.axis_index("c")
    n0 = wid * _NPW

    zeros = jnp.zeros((_L,), jnp.float32)
    ones = jnp.ones((_L,), jnp.float32)
    lane = lax.iota(jnp.int32, _L)

    pltpu.sync_copy(x_hbm.at[pl.ds(n0 * _XSLAB, _XW)], xv)

    for buf in (buf0, buf1):
        def zrow(t, carry, buf=buf):
            def zcol(j, c):
                off = jnp.minimum(j * _L, _V - _L)
                buf[t, pl.ds(off, _L)] = zeros
                return c

            return lax.fori_loop(0, _ZVEC, zcol, carry)

        lax.fori_loop(0, _T, zrow, 0)

    def scatter_slab(buf, k, val):
        xoff = k * _XSLAB

        def body(i, carry):
            e = jnp.minimum(i * _L, _XSLAB - _L) + lane
            t = lax.shift_right_logical(e * 52429, 19)
            v = xv[pl.ds(jnp.minimum(i * _L, _XSLAB - _L) + xoff, _L)]
            plsc.store_scatter(buf, [t, v], val)
            return carry

        lax.fori_loop(0, _SVEC, body, 0)

    def issue_out(buf, sem, k):
        pltpu.async_copy(buf, out_hbm.at[n0 + k], sem)

    for b, (buf, sem) in enumerate(((buf0, sem0), (buf1, sem1))):
        scatter_slab(buf, b, ones)
        issue_out(buf, sem, b)

    def pair_body(j, carry):
        for b, (buf, sem) in enumerate(((buf0, sem0), (buf1, sem1))):
            k = 2 * j + b
            pltpu.make_async_copy(buf, out_hbm.at[0], sem).wait()
            scatter_slab(buf, k - 2, zeros)
            scatter_slab(buf, k, ones)
            issue_out(buf, sem, k)
        return carry

    lax.fori_loop(1, _NPW // 2, pair_body, 0)

    for buf, sem in ((buf0, sem0), (buf1, sem1)):
        pltpu.make_async_copy(buf, out_hbm.at[0], sem).wait()


def kernel(x):
    return _ten_hot(x.reshape(-1))


# speedup vs baseline: 6.7344x; 1.0003x over previous
import functools

import jax
import jax.numpy as jnp
from jax import lax
from jax.experimental import pallas as pl
from jax.experimental.pallas import tpu as pltpu
from jax.experimental.pallas import tpu_sc as plsc

_N = 1024
_T = 50
_V = 1000
_F = 10

_info = plsc.get_sparse_core_info()
_NC, _NS, _L = _info.num_cores, _info.num_subcores, _info.num_lanes
_NW = _NC * _NS
_NPW = _N // _NW
_XSLAB = _T * _F
_XW = _NPW * _XSLAB
_SVEC = (_XSLAB + _L - 1) // _L
_ZVEC = (_V + _L - 1) // _L

_mesh = plsc.VectorSubcoreMesh(core_axis_name="c", subcore_axis_name="s")


@functools.partial(
    pl.kernel,
    out_type=jax.ShapeDtypeStruct((_N, _T, _V), jnp.float32),
    mesh=_mesh,
    scratch_types=[
        pltpu.VMEM((_T, _V), jnp.float32),
        pltpu.VMEM((_T, _V), jnp.float32),
        pltpu.VMEM((_XW,), jnp.int32),
        pltpu.SemaphoreType.DMA,
        pltpu.SemaphoreType.DMA,
    ],
    compiler_params=pltpu.CompilerParams(
        needs_layout_passes=False, use_tc_tiling_on_sc=True
    ),
)
def _ten_hot(x_hbm, out_hbm, buf0, buf1, xv, sem0, sem1):
    wid = lax.axis_index("s") * _NC + lax.axis_index("c")
    n0 = wid * _NPW

    zeros = jnp.zeros((_L,), jnp.float32)
    ones = jnp.ones((_L,), jnp.float32)
    lane = lax.iota(jnp.int32, _L)

    pltpu.sync_copy(x_hbm.at[pl.ds(n0 * _XSLAB, _XW)], xv)

    for buf in (buf0, buf1):
        def zrow(t, carry, buf=buf):
            def zcol(j, c):
                off = jnp.minimum(j * _L, _V - _L)
                buf[t, pl.ds(off, _L)] = zeros
                return c

            return lax.fori_loop(0, _ZVEC, zcol, carry)

        lax.fori_loop(0, _T, zrow, 0)

    def scatter_slab(buf, k, val):
        xoff = k * _XSLAB

        def body(i, carry):
            e = jnp.minimum(i * _L, _XSLAB - _L) + lane
            t = lax.shift_right_logical(e * 52429, 19)
            v = xv[pl.ds(jnp.minimum(i * _L, _XSLAB - _L) + xoff, _L)]
            plsc.store_scatter(buf, [t, v], val)
            return carry

        lax.fori_loop(0, _SVEC, body, 0)

    def issue_out(buf, sem, k):
        pltpu.async_copy(buf, out_hbm.at[n0 + k], sem)

    for b, (buf, sem) in enumerate(((buf0, sem0), (buf1, sem1))):
        scatter_slab(buf, b, ones)
        issue_out(buf, sem, b)

    def pair_body(j, carry):
        for b, (buf, sem) in enumerate(((buf0, sem0), (buf1, sem1))):
            k = 2 * j + b
            pltpu.make_async_copy(buf, out_hbm.at[0], sem).wait()
            scatter_slab(buf, k - 2, zeros)
            scatter_slab(buf, k, ones)
            issue_out(buf, sem, k)
        return carry

    lax.fori_loop(1, _NPW // 2, pair_body, 0)

    for buf, sem in ((buf0, sem0), (buf1, sem1)):
        pltpu.make_async_copy(buf, out_hbm.at[0], sem).wait()


def kernel(x):
    return _ten_hot(x.reshape(-1))
